# Initial kernel scaffold; baseline (speedup 1.0000x reference)
import jax
import jax.numpy as jnp
from jax.experimental import pallas as pl


def kernel(test_input, emb_table):
    raise NotImplementedError("write your pallas kernel here")


# speedup vs baseline: 1.4007x; 1.4007x over previous
import functools

import jax
import jax.numpy as jnp
from jax import lax
from jax.experimental import pallas as pl
from jax.experimental.pallas import tpu as pltpu
from jax.experimental.pallas import tpu_sc as plsc

_VOCAB = 32128
_DIM = 4096
_B = 4
_S = 512
_N = _B * _S
_NC = 2
_NS = 16
_NW = _NC * _NS
_RPW = _N // _NW
_R = 8
_NCHUNK = _RPW // _R

_mesh = plsc.VectorSubcoreMesh(core_axis_name="c", subcore_axis_name="s")


@functools.partial(
    pl.kernel,
    out_type=jax.ShapeDtypeStruct((_N, _DIM), jnp.float32),
    mesh=_mesh,
    scratch_types=[
        pltpu.VMEM((_NCHUNK, _R), jnp.int32),
        pltpu.VMEM((2, _R, _DIM), jnp.float32),
        pltpu.SemaphoreType.DMA,
    ],
)
def _emb_lookup(table_hbm, idx_hbm, out_hbm, idx_v, rows_v, sem):
    wid = lax.axis_index("s") * _NC + lax.axis_index("c")
    pltpu.sync_copy(idx_hbm.at[pl.ds(wid * _NCHUNK, _NCHUNK)], idx_v)
    for c in range(_NCHUNK):
        buf = rows_v.at[c % 2]
        pltpu.async_copy(table_hbm.at[idx_v.at[c]], buf, sem).wait()
        pltpu.sync_copy(buf, out_hbm.at[pl.ds(wid * _RPW + c * _R, _R)])


def kernel(test_input, emb_table):
    idx = test_input.reshape(_NW * _NCHUNK, _R)
    out = _emb_lookup(emb_table, idx)
    return out.reshape(_B, _S, _DIM)


# speedup vs baseline: 1.5356x; 1.0963x over previous
import functools

import jax
import jax.numpy as jnp
from jax import lax
from jax.experimental import pallas as pl
from jax.experimental.pallas import tpu as pltpu
from jax.experimental.pallas import tpu_sc as plsc

_VOCAB = 32128
_DIM = 4096
_B = 4
_S = 512
_N = _B * _S
_NC = 2
_NS = 16
_NW = _NC * _NS
_RPW = _N // _NW
_R = 8
_NCHUNK = _RPW // _R
_NBUF = 3

_mesh = plsc.VectorSubcoreMesh(core_axis_name="c", subcore_axis_name="s")


@functools.partial(
    pl.kernel,
    out_type=jax.ShapeDtypeStruct((_N, _DIM), jnp.float32),
    mesh=_mesh,
    scratch_types=[
        pltpu.VMEM((_NCHUNK, _R), jnp.int32),
        pltpu.VMEM((_NBUF, _R, _DIM), jnp.float32),
        [pltpu.SemaphoreType.DMA] * _NBUF,
        [pltpu.SemaphoreType.DMA] * _NBUF,
    ],
)
def _emb_lookup(table_hbm, idx_hbm, out_hbm, idx_v, rows_v, gsems, osems):
    wid = lax.axis_index("s") * _NC + lax.axis_index("c")
    pltpu.sync_copy(idx_hbm.at[pl.ds(wid * _NCHUNK, _NCHUNK)], idx_v)

    def gather(c):
        b = c % _NBUF
        return pltpu.async_copy(table_hbm.at[idx_v.at[c]], rows_v.at[b], gsems[b])

    def put(c):
        b = c % _NBUF
        return pltpu.async_copy(
            rows_v.at[b], out_hbm.at[pl.ds(wid * _RPW + c * _R, _R)], osems[b]
        )

    gd = [gather(c) for c in range(_NBUF - 1)] + [None] * (_NCHUNK - _NBUF + 1)
    od = [None] * _NCHUNK
    for c in range(_NCHUNK):
        gd[c].wait()
        od[c] = put(c)
        nxt = c + _NBUF - 1
        if nxt < _NCHUNK:
            if nxt - _NBUF >= 0:
                od[nxt - _NBUF].wait()
            gd[nxt] = gather(nxt)
    for c in range(_NCHUNK - _NBUF, _NCHUNK):
        od[c].wait()


def kernel(test_input, emb_table):
    idx = test_input.reshape(_NW * _NCHUNK, _R)
    out = _emb_lookup(emb_table, idx)
    return out.reshape(_B, _S, _DIM)


# speedup vs baseline: 1.5365x; 1.0006x over previous
import functools

import jax
import jax.numpy as jnp
from jax import lax
from jax.experimental import pallas as pl
from jax.experimental.pallas import tpu as pltpu
from jax.experimental.pallas import tpu_sc as plsc

_VOCAB = 32128
_DIM = 4096
_B = 4
_S = 512
_N = _B * _S
_NC = 2
_NS = 16
_NW = _NC * _NS
_RPW = _N // _NW
_WPB = _S // _RPW
_R = 8
_NCHUNK = _RPW // _R
_NBUF = 3

_mesh = plsc.VectorSubcoreMesh(core_axis_name="c", subcore_axis_name="s")


@functools.partial(
    pl.kernel,
    out_type=jax.ShapeDtypeStruct((_B, _S, _DIM), jnp.float32),
    mesh=_mesh,
    scratch_types=[
        pltpu.VMEM((_RPW,), jnp.int32),
        pltpu.VMEM((_NBUF, _R, _DIM), jnp.float32),
        [pltpu.SemaphoreType.DMA] * _NBUF,
        [pltpu.SemaphoreType.DMA] * _NBUF,
    ],
)
def _emb_lookup(table_hbm, idx_hbm, out_hbm, idx_v, rows_v, gsems, osems):
    wid = lax.axis_index("s") * _NC + lax.axis_index("c")
    b = wid // _WPB
    s0 = (wid % _WPB) * _RPW
    pltpu.sync_copy(idx_hbm.at[b, pl.ds(s0, _RPW)], idx_v)

    def gather(c):
        return pltpu.async_copy(
            table_hbm.at[idx_v.at[pl.ds(c * _R, _R)]],
            rows_v.at[c % _NBUF],
            gsems[c % _NBUF],
        )

    def put(c):
        return pltpu.async_copy(
            rows_v.at[c % _NBUF],
            out_hbm.at[b, pl.ds(s0 + c * _R, _R)],
            osems[c % _NBUF],
        )

    gd = [gather(c) for c in range(_NBUF - 1)] + [None] * (_NCHUNK - _NBUF + 1)
    od = [None] * _NCHUNK
    for c in range(_NCHUNK):
        gd[c].wait()
        od[c] = put(c)
        nxt = c + _NBUF - 1
        if nxt < _NCHUNK:
            if nxt - _NBUF >= 0:
                od[nxt - _NBUF].wait()
            gd[nxt] = gather(nxt)
    for c in range(_NCHUNK - _NBUF, _NCHUNK):
        od[c].wait()


def kernel(test_input, emb_table):
    return _emb_lookup(emb_table, test_input)
